# baseline (speedup 1.0000x reference)
import jax
import jax.numpy as jnp
from jax import lax
from jax.experimental import pallas as pl
from jax.experimental.pallas import tpu as pltpu
from jax.experimental.pallas import tpu_sc as plsc

_ROWS = 100000
_COLS = 256
_NC, _NS = 2, 16
_NW = _NC * _NS
_CHUNK_ROWS = 160
_NBUF = 3
_STEPS = 19
_MAIN_CHUNKS = _NW * _STEPS
_TAIL_CHUNKS = _ROWS // _CHUNK_ROWS - _MAIN_CHUNKS


def _sc_copy_body(src, dst, b0, b1, b2, si0, si1, si2, so0, so1, so2):
    cid = lax.axis_index("c")
    sid = lax.axis_index("s")
    wid = cid * _NS + sid
    bufs = (b0, b1, b2)
    in_sems = (si0, si1, si2)
    out_sems = (so0, so1, so2)

    def row0(k):
        return (wid + k * _NW) * _CHUNK_ROWS

    def start_in(k):
        i = k % _NBUF
        return pltpu.async_copy(
            src.at[pl.ds(row0(k), _CHUNK_ROWS)], bufs[i], in_sems[i])

    def start_out(k):
        i = k % _NBUF
        return pltpu.async_copy(
            bufs[i], dst.at[pl.ds(row0(k), _CHUNK_ROWS)], out_sems[i])

    in_p = {k: start_in(k) for k in range(_NBUF)}
    out_p = {}
    waited = -1
    for k in range(_STEPS):
        in_p[k].wait()
        out_p[k] = start_out(k)
        if k >= 1:
            out_p[k - 1].wait()
            waited = k - 1
            nk = k + 2
            if _NBUF <= nk < _STEPS:
                in_p[nk] = start_in(nk)
    for k in range(waited + 1, _STEPS):
        out_p[k].wait()

    j = sid * _NC + cid

    @pl.when(j < _TAIL_CHUNKS)
    def _():
        r0 = (_MAIN_CHUNKS + j) * _CHUNK_ROWS
        pltpu.sync_copy(src.at[pl.ds(r0, _CHUNK_ROWS)], b0)
        pltpu.sync_copy(b0, dst.at[pl.ds(r0, _CHUNK_ROWS)])


def kernel(weight):
    mesh = plsc.VectorSubcoreMesh(core_axis_name="c", subcore_axis_name="s")
    return pl.kernel(
        _sc_copy_body,
        out_type=jax.ShapeDtypeStruct((_ROWS, _COLS), jnp.float32),
        mesh=mesh,
        scratch_types=[
            pltpu.VMEM((_CHUNK_ROWS, _COLS), jnp.float32),
            pltpu.VMEM((_CHUNK_ROWS, _COLS), jnp.float32),
            pltpu.VMEM((_CHUNK_ROWS, _COLS), jnp.float32),
            pltpu.SemaphoreType.DMA,
            pltpu.SemaphoreType.DMA,
            pltpu.SemaphoreType.DMA,
            pltpu.SemaphoreType.DMA,
            pltpu.SemaphoreType.DMA,
            pltpu.SemaphoreType.DMA,
        ],
    )(weight)


# speedup vs baseline: 1.0501x; 1.0501x over previous
import jax
import jax.numpy as jnp
from jax import lax
from jax.experimental import pallas as pl
from jax.experimental.pallas import tpu as pltpu
from jax.experimental.pallas import tpu_sc as plsc

_ROWS = 100000
_COLS = 256
_NC, _NS = 2, 16
_NW = _NC * _NS
_CHUNK_ROWS = 240
_STEPS = 13
_TAIL_BASE = _NW * _STEPS * _CHUNK_ROWS
_TAIL_PIECES = 4
_TAIL_ROWS = (_ROWS - _TAIL_BASE) // _TAIL_PIECES


def _sc_copy_body(src, dst, b0, b1, si0, si1, so0, so1):
    cid = lax.axis_index("c")
    sid = lax.axis_index("s")
    wid = cid * _NS + sid
    bufs = (b0, b1)
    in_sems = (si0, si1)
    out_sems = (so0, so1)

    def row0(k):
        return (wid + k * _NW) * _CHUNK_ROWS

    def start_in(k):
        i = k % 2
        return pltpu.async_copy(
            src.at[pl.ds(row0(k), _CHUNK_ROWS)], bufs[i], in_sems[i])

    def start_out(k):
        i = k % 2
        return pltpu.async_copy(
            bufs[i], dst.at[pl.ds(row0(k), _CHUNK_ROWS)], out_sems[i])

    in_p = [start_in(0), start_in(1)]
    out_p = [None, None]
    for k in range(_STEPS):
        i = k % 2
        in_p[i].wait()
        out_p[i] = start_out(k)
        nk = k + 2
        if nk < _STEPS:
            out_p[i].wait()
            in_p[i] = start_in(nk)

    p = (sid - (_NS - 2)) * _NC + cid
    is_tail = (sid >= _NS - 2) & (p < _TAIL_PIECES)
    out_p[(_STEPS - 2) % 2].wait()

    @pl.when(is_tail)
    def _():
        r0 = _TAIL_BASE + p * _TAIL_ROWS
        tb = b1.at[pl.ds(0, _TAIL_ROWS)]
        pltpu.async_copy(src.at[pl.ds(r0, _TAIL_ROWS)], tb, si1).wait()
        pltpu.async_copy(tb, dst.at[pl.ds(r0, _TAIL_ROWS)], so1).wait()

    out_p[(_STEPS - 1) % 2].wait()


def kernel(weight):
    mesh = plsc.VectorSubcoreMesh(core_axis_name="c", subcore_axis_name="s")
    return pl.kernel(
        _sc_copy_body,
        out_type=jax.ShapeDtypeStruct((_ROWS, _COLS), jnp.float32),
        mesh=mesh,
        scratch_types=[
            pltpu.VMEM((_CHUNK_ROWS, _COLS), jnp.float32),
            pltpu.VMEM((_CHUNK_ROWS, _COLS), jnp.float32),
            pltpu.SemaphoreType.DMA,
            pltpu.SemaphoreType.DMA,
            pltpu.SemaphoreType.DMA,
            pltpu.SemaphoreType.DMA,
        ],
    )(weight)
